# Initial kernel scaffold; baseline (speedup 1.0000x reference)
import jax
import jax.numpy as jnp
from jax.experimental import pallas as pl


def kernel(values, value_embed_w, row_embed_w, col_embed_w, box_embed_w, input_bias, ln_gamma, ln_beta):
    raise NotImplementedError("write your pallas kernel here")


# speedup vs baseline: 3.9653x; 3.9653x over previous
import functools

import jax
import jax.numpy as jnp
from jax import lax
from jax.experimental import pallas as pl
from jax.experimental.pallas import tpu as pltpu

BATCH = 1024
SEQ = 81
VOCAB = 11
GRID = 9
H = 512
VPAD = 16
EPS = 1e-5

BB = 128


def _table_body(vw_ref, rw_ref, cw_ref, bw_ref, bias_ref, g_ref, b_ref, out_ref):
    n = SEQ * VPAD
    r = lax.broadcasted_iota(jnp.int32, (n, 1), 0)
    s_idx = r // VPAD
    v_idx = r % VPAD
    row_ids = s_idx // GRID
    col_ids = s_idx % GRID
    box_ids = (row_ids // 3) * 3 + (col_ids // 3)

    oh_v = (v_idx == lax.broadcasted_iota(jnp.int32, (n, VOCAB), 1)).astype(jnp.float32)
    oh_r = (row_ids == lax.broadcasted_iota(jnp.int32, (n, GRID), 1)).astype(jnp.float32)
    oh_c = (col_ids == lax.broadcasted_iota(jnp.int32, (n, GRID), 1)).astype(jnp.float32)
    oh_b = (box_ids == lax.broadcasted_iota(jnp.int32, (n, GRID), 1)).astype(jnp.float32)

    x = jnp.dot(oh_v, vw_ref[...], preferred_element_type=jnp.float32)
    x = x + jnp.dot(oh_r, rw_ref[...], preferred_element_type=jnp.float32)
    x = x + jnp.dot(oh_c, cw_ref[...], preferred_element_type=jnp.float32)
    x = x + jnp.dot(oh_b, bw_ref[...], preferred_element_type=jnp.float32)
    x = x + bias_ref[...]

    mean = jnp.mean(x, axis=1, keepdims=True)
    xc = x - mean
    var = jnp.mean(xc * xc, axis=1, keepdims=True)
    y = xc / jnp.sqrt(var + EPS) * g_ref[...] + b_ref[...]
    out_ref[...] = y


def _build_table(vw, rw, cw, bw, bias, gamma, beta):
    return pl.pallas_call(
        _table_body,
        out_shape=jax.ShapeDtypeStruct((SEQ * VPAD, H), jnp.float32),
    )(vw, rw, cw, bw, bias.reshape(1, H), gamma.reshape(1, H), beta.reshape(1, H))


def _gather_body(values_ref, t_ref, out_ref):
    for s in range(SEQ):
        vcol = values_ref[:, s : s + 1]
        oh = (vcol == lax.broadcasted_iota(jnp.int32, (BB, VPAD), 1)).astype(jnp.float32)
        ts = t_ref[s]
        out_ref[:, s, :] = jnp.dot(oh, ts, preferred_element_type=jnp.float32)


def _gather(values, table3):
    grid = BATCH // BB
    return pl.pallas_call(
        _gather_body,
        grid=(grid,),
        in_specs=[
            pl.BlockSpec((BB, SEQ), lambda i: (i, 0)),
            pl.BlockSpec((SEQ, VPAD, H), lambda i: (0, 0, 0)),
        ],
        out_specs=pl.BlockSpec((BB, SEQ, H), lambda i: (i, 0, 0)),
        out_shape=jax.ShapeDtypeStruct((BATCH, SEQ, H), jnp.float32),
        compiler_params=pltpu.CompilerParams(
            dimension_semantics=("parallel",),
        ),
    )(values, table3)


def kernel(values, value_embed_w, row_embed_w, col_embed_w, box_embed_w, input_bias, ln_gamma, ln_beta):
    t2d = _build_table(value_embed_w, row_embed_w, col_embed_w, box_embed_w,
                       input_bias, ln_gamma, ln_beta)
    t3 = t2d.reshape(SEQ, VPAD, H)
    return _gather(values.astype(jnp.int32), t3)
